# Initial kernel scaffold; baseline (speedup 1.0000x reference)
import jax
import jax.numpy as jnp
from jax.experimental import pallas as pl


def kernel(x, edge_index, batch, W1, b1, W2, b2, p1_wrel, p1_wroot, p1_b, p2_wrel, p2_wroot, p2_b):
    raise NotImplementedError("write your pallas kernel here")


# speedup vs baseline: 75.4938x; 75.4938x over previous
import functools

import jax
import jax.numpy as jnp
from jax import lax
from jax.experimental import pallas as pl
from jax.experimental.pallas import tpu as pltpu
from jax.experimental.pallas import tpu_sc as plsc

N = 10000
E = 320000
D_FEAT = 128
HIDDEN = 16
NUM_CLASSES = 10
K1 = 2500
K2 = 625

NP = 10240
SPR = NP // 16
NTILES = 32
EPT = E // NTILES
CH = 125
NCH = EPT // CH
GRP = EPT // 16


def _sc_scalar_seg_body(src_hbm, row_hbm, col_hbm, out_hbm, srcv, rbuf, cbuf, acc):
    cid = lax.axis_index("c")
    sid = lax.axis_index("s")
    w = sid * 2 + cid
    base = w * EPT
    pltpu.sync_copy(src_hbm, srcv)
    pltpu.sync_copy(row_hbm.at[pl.ds(base, EPT)], rbuf)
    pltpu.sync_copy(col_hbm.at[pl.ds(base, EPT)], cbuf)

    zero = jnp.zeros((16,), jnp.float32)

    def zbody(j, carry):
        acc[pl.ds(j * 16, 16)] = zero
        return carry

    lax.fori_loop(0, GRP, zbody, 0)

    def body(j, carry):
        r = rbuf[pl.ds(j * 16, 16)]
        c = cbuf[pl.ds(j * 16, 16)]
        vals = plsc.load_gather(srcv, [r])
        plsc.addupdate_scatter(acc, [c], vals)
        return carry

    lax.fori_loop(0, GRP, body, 0)
    pltpu.sync_copy(acc, out_hbm.at[w])


def _sc_vec_seg_body(rows_hbm, row3_hbm, col3_hbm, zeros_hbm, out_hbm,
                     rbuf, cbuf, rowsv, spacc, sem):
    cid = lax.axis_index("c")
    sid = lax.axis_index("s")
    w = sid * 2 + cid
    pltpu.sync_copy(row3_hbm.at[w], rbuf)
    pltpu.sync_copy(col3_hbm.at[w], cbuf)
    sl = pl.ds(sid * SPR, SPR)
    pltpu.sync_copy(zeros_hbm.at[sl], spacc.at[sl])
    plsc.subcore_barrier()

    def body(j, carry):
        pltpu.async_copy(rows_hbm.at[rbuf.at[j]], rowsv, sem).wait()
        pltpu.sync_copy(rowsv, spacc.at[cbuf.at[j]], add=True)
        return carry

    lax.fori_loop(0, NCH, body, 0)
    plsc.subcore_barrier()
    pltpu.sync_copy(spacc.at[sl], out_hbm.at[cid].at[sl])


@functools.lru_cache(maxsize=None)
def _sc_kernels():
    mesh = plsc.VectorSubcoreMesh(core_axis_name="c", subcore_axis_name="s")
    params = pltpu.CompilerParams(needs_layout_passes=False,
                                  use_tc_tiling_on_sc=False)
    scalar = pl.kernel(
        _sc_scalar_seg_body,
        mesh=mesh,
        compiler_params=params,
        out_type=jax.ShapeDtypeStruct((NTILES, N), jnp.float32),
        scratch_types=[
            pltpu.VMEM((N,), jnp.float32),
            pltpu.VMEM((EPT,), jnp.int32),
            pltpu.VMEM((EPT,), jnp.int32),
            pltpu.VMEM((N,), jnp.float32),
        ],
    )
    vec = pl.kernel(
        _sc_vec_seg_body,
        mesh=mesh,
        compiler_params=params,
        out_type=jax.ShapeDtypeStruct((2, NP, HIDDEN), jnp.float32),
        scratch_types=[
            pltpu.VMEM((NCH, CH), jnp.int32),
            pltpu.VMEM((NCH, CH), jnp.int32),
            pltpu.VMEM((CH, HIDDEN), jnp.float32),
            pltpu.VMEM_SHARED((NP, HIDDEN), jnp.float32),
            pltpu.SemaphoreType.DMA,
        ],
    )
    return scalar, vec


def _topk_mask(score, valid, k, iota_dim):
    imin = jnp.int32(-2147483648)
    b = lax.bitcast_convert_type(score, jnp.int32)
    v = jnp.where(b >= 0, b, imin - b)

    def cnt(pred):
        return jnp.sum(jnp.where(pred & valid, jnp.int32(1), jnp.int32(0)))

    t0 = jnp.where(cnt(v >= 0) >= k, jnp.int32(0), imin)

    def bit_body(i, t):
        tt = t | (jnp.int32(1) << (jnp.int32(30) - i))
        return jnp.where(cnt(v >= tt) >= k, tt, t)

    t = lax.fori_loop(0, 31, bit_body, t0)
    cnt_gt = cnt(v > t)
    need = jnp.int32(k) - cnt_gt
    eq = valid & (v == t)
    idx = lax.broadcasted_iota(jnp.int32, score.shape, iota_dim)

    def ib(i, lohi):
        lo, hi = lohi
        mid = (lo + hi) // 2
        c = jnp.sum(jnp.where(eq & (idx <= mid), jnp.int32(1), jnp.int32(0)))
        ok = c >= need
        return jnp.where(ok, lo, mid + 1), jnp.where(ok, mid, hi)

    lo, _ = lax.fori_loop(0, 14, ib, (jnp.int32(0), jnp.int32(N - 1)))
    return (valid & (v > t)) | (eq & (idx <= lo))


def _eye16():
    r = lax.broadcasted_iota(jnp.int32, (HIDDEN, HIDDEN), 0)
    c = lax.broadcasted_iota(jnp.int32, (HIDDEN, HIDDEN), 1)
    return jnp.where(r == c, 1.0, 0.0).astype(jnp.float32)


def _dg(a, b, ca, cb):
    return lax.dot_general(a, b, (((ca,), (cb,)), ((), ())),
                           preferred_element_type=jnp.float32)


def _to_fmajor(m):
    return _dg(_eye16(), m, 1, 1)


def _to_nmajor(m):
    return _dg(m, _eye16(), 0, 0)


def _tc(f, out_shape, *args):
    return pl.pallas_call(f, out_shape=out_shape)(*args)


def _tc_a(x, W1, degp):
    def f(x_r, w_r, dp_r, hs_r, dinv_r):
        deg = jnp.sum(dp_r[...], axis=0, keepdims=True) + 1.0
        dinv = lax.rsqrt(deg)
        hT = _dg(w_r[...], x_r[...], 0, 1)
        hs_r[...] = _to_nmajor(hT * dinv)
        dinv_r[...] = dinv

    return _tc(f, [jax.ShapeDtypeStruct((N, HIDDEN), jnp.float32),
                   jax.ShapeDtypeStruct((1, N), jnp.float32)],
               x, W1, degp)


def _tc_b(aggp, hs, dinv1, b1, p1r, p1o, p1b):
    def f(ap_r, hs_r, di_r, b1_r, pr_r, po_r, pb_r, h1_r, srel_r, base_r):
        agg = jnp.sum(ap_r[...], axis=0)[:N] + hs_r[...]
        mT = _to_fmajor(agg)
        h1T = jnp.maximum(di_r[...] * mT + b1_r[...], 0.0)
        h1_r[...] = _to_nmajor(h1T)
        srel_r[...] = _dg(pr_r[...], h1T, 0, 0)
        base_r[...] = _dg(po_r[...], h1T, 0, 0) + pb_r[...]

    return _tc(f, [jax.ShapeDtypeStruct((N, HIDDEN), jnp.float32),
                   jax.ShapeDtypeStruct((1, N), jnp.float32),
                   jax.ShapeDtypeStruct((1, N), jnp.float32)],
               aggp, hs, dinv1, b1, p1r, p1o, p1b)


def _tc_c(s1p, base1, h1, W2p):
    def f(sp_r, ba_r, h1_r, w2_r, self_r, z_r):
        score = jnp.sum(sp_r[...], axis=0, keepdims=True) + ba_r[...]
        sel = _topk_mask(score, score > jnp.float32(-jnp.inf), K1, 1)
        m1 = jnp.where(sel, jnp.tanh(score), 0.0)
        xpT = _to_fmajor(h1_r[...]) * m1
        zT = _dg(w2_r[...], xpT, 0, 0)
        self_r[...] = jnp.where(sel, 1.0, 0.0)
        z_r[...] = _to_nmajor(zT)

    return _tc(f, [jax.ShapeDtypeStruct((1, N), jnp.float32),
                   jax.ShapeDtypeStruct((N, HIDDEN), jnp.float32)],
               s1p, base1, h1, W2p)


def _tc_d(d2p, z16):
    def f(dp_r, z_r, zs_r, dinv_r):
        deg2 = jnp.sum(dp_r[...], axis=0, keepdims=True) + 1.0
        dinv2 = lax.rsqrt(deg2)
        zs_r[...] = _to_nmajor(_to_fmajor(z_r[...]) * dinv2)
        dinv_r[...] = dinv2

    return _tc(f, [jax.ShapeDtypeStruct((N, HIDDEN), jnp.float32),
                   jax.ShapeDtypeStruct((1, N), jnp.float32)],
               d2p, z16)


def _tc_e(a2p, zs16, dinv2, b2p, p2r, p2o, p2b, sel1f):
    def f(ap_r, zs_r, di_r, b2_r, pr_r, po_r, pb_r, sf_r, h2_r, srel_r, base_r):
        agg2 = jnp.sum(ap_r[...], axis=0)[:N] + zs_r[...]
        h2T = di_r[...] * _to_fmajor(agg2) + b2_r[...]
        h2_r[...] = h2T
        srel_r[...] = _dg(pr_r[...], h2T, 0, 0) * sf_r[...]
        base_r[...] = _dg(po_r[...], h2T, 0, 0) + pb_r[...]

    return _tc(f, [jax.ShapeDtypeStruct((HIDDEN, N), jnp.float32),
                   jax.ShapeDtypeStruct((1, N), jnp.float32),
                   jax.ShapeDtypeStruct((1, N), jnp.float32)],
               a2p, zs16, dinv2, b2p, p2r, p2o, p2b, sel1f)


def _tc_f(s2p, base2, h2T, sel1f):
    def f(sp_r, ba_r, h2_r, sf_r, out_r):
        score2 = jnp.sum(sp_r[...], axis=0, keepdims=True) + ba_r[...]
        sel2 = _topk_mask(score2, sf_r[...] > 0.5, K2, 1)
        m2 = jnp.where(sel2, jnp.tanh(score2), 0.0)
        gcol = jnp.sum(jnp.where(sel2, h2_r[...] * m2, 0.0), axis=1,
                       keepdims=True)
        grow = _dg(gcol, _eye16(), 0, 0)[:, :NUM_CLASSES] * jnp.float32(1.0 / K2)
        mx = jnp.max(grow, axis=1, keepdims=True)
        sh = grow - mx
        lse = jnp.log(jnp.sum(jnp.exp(sh), axis=1, keepdims=True))
        out_r[...] = sh - lse

    return _tc(f, [jax.ShapeDtypeStruct((1, NUM_CLASSES), jnp.float32)],
               s2p, base2, h2T, sel1f)[0]


def kernel(x, edge_index, batch, W1, b1, W2, b2,
           p1_wrel, p1_wroot, p1_b, p2_wrel, p2_wroot, p2_b):
    row = edge_index[0]
    col = edge_index[1]
    row3 = row.reshape(NTILES, NCH, CH)
    col3 = col.reshape(NTILES, NCH, CH)
    zeros16 = jnp.zeros((NP, HIDDEN), jnp.float32)
    ones_n = jnp.ones((N,), jnp.float32)

    b1r = b1.reshape(HIDDEN, 1)
    W2p = jnp.zeros((HIDDEN, HIDDEN), jnp.float32).at[:, :NUM_CLASSES].set(W2)
    b2p = jnp.zeros((HIDDEN, 1), jnp.float32).at[:NUM_CLASSES, 0].set(b2)
    p2rp = jnp.zeros((HIDDEN, 1), jnp.float32).at[:NUM_CLASSES].set(p2_wrel)
    p2op = jnp.zeros((HIDDEN, 1), jnp.float32).at[:NUM_CLASSES].set(p2_wroot)

    sc_scalar, sc_vec = _sc_kernels()

    degp = sc_scalar(ones_n, row, col)
    hs, dinv1 = _tc_a(x, W1, degp)
    aggp = sc_vec(hs, row3, col3, zeros16)
    h1, s1rel, base1 = _tc_b(aggp, hs, dinv1, b1r, p1_wrel, p1_wroot,
                             p1_b.reshape(1, 1))
    s1p = sc_scalar(s1rel.reshape(N), row, col)
    sel1f, z16 = _tc_c(s1p, base1, h1, W2p)
    d2p = sc_scalar(sel1f.reshape(N), row, col)
    zs16, dinv2 = _tc_d(d2p, z16)
    a2p = sc_vec(zs16, row3, col3, zeros16)
    h2T, s2rel, base2 = _tc_e(a2p, zs16, dinv2, b2p, p2rp, p2op,
                              p2_b.reshape(1, 1), sel1f)
    s2p = sc_scalar(s2rel.reshape(N), row, col)
    return _tc_f(s2p, base2, h2T, sel1f)


# speedup vs baseline: 108.2967x; 1.4345x over previous
import functools

import jax
import jax.numpy as jnp
from jax import lax
from jax.experimental import pallas as pl
from jax.experimental.pallas import tpu as pltpu
from jax.experimental.pallas import tpu_sc as plsc

N = 10000
E = 320000
D_FEAT = 128
HIDDEN = 16
NUM_CLASSES = 10
K1 = 2500
K2 = 625

NP = 10240
SPR = NP // 16
NTILES = 32
EPT = E // NTILES
CH = 125
NCH = EPT // CH
FIRE = 4
GRP = EPT // 16


def _sc_scalar_seg_body(src_hbm, row_hbm, col_hbm, out_hbm, srcv, rbuf, cbuf, acc):
    cid = lax.axis_index("c")
    sid = lax.axis_index("s")
    w = sid * 2 + cid
    base = w * EPT
    pltpu.sync_copy(src_hbm, srcv)
    pltpu.sync_copy(row_hbm.at[pl.ds(base, EPT)], rbuf)
    pltpu.sync_copy(col_hbm.at[pl.ds(base, EPT)], cbuf)

    zero = jnp.zeros((16,), jnp.float32)

    def zbody(j5, carry):
        for u in range(5):
            acc[pl.ds((j5 * 5 + u) * 16, 16)] = zero
        return carry

    lax.fori_loop(0, GRP // 5, zbody, 0)

    def body(j5, carry):
        for u in range(5):
            j = j5 * 5 + u
            r = rbuf[pl.ds(j * 16, 16)]
            c = cbuf[pl.ds(j * 16, 16)]
            vals = plsc.load_gather(srcv, [r])
            plsc.addupdate_scatter(acc, [c], vals)
        return carry

    lax.fori_loop(0, GRP // 5, body, 0)
    pltpu.sync_copy(acc, out_hbm.at[w])


def _sc_vec_seg_body(rows_hbm, row3_hbm, col3_hbm, zeros_hbm, out_hbm,
                     rbuf, cbuf, rva, rvb, spacc, sga, sgb, ssc):
    cid = lax.axis_index("c")
    sid = lax.axis_index("s")
    w = sid * 2 + cid
    pltpu.sync_copy(row3_hbm.at[w], rbuf)
    pltpu.sync_copy(col3_hbm.at[w], cbuf)
    sl = pl.ds(sid * SPR, SPR)
    pltpu.sync_copy(zeros_hbm.at[sl], spacc.at[sl])
    plsc.subcore_barrier()

    def fire(g, rv, sem):
        return [pltpu.async_copy(rows_hbm.at[rbuf.at[g * FIRE + b]],
                                 rv.at[pl.ds(b * CH, CH)], sem)
                for b in range(FIRE)]

    def scatter(g, rv):
        return [pltpu.async_copy(rv.at[pl.ds(b * CH, CH)],
                                 spacc.at[cbuf.at[g * FIRE + b]], ssc, add=True)
                for b in range(FIRE)]

    def pair(i2, carry):
        g0 = i2 * 2
        ga = fire(g0, rva, sga)
        gb = fire(g0 + 1, rvb, sgb)
        for cp in ga:
            cp.wait()
        sa = scatter(g0, rva)
        for cp in gb:
            cp.wait()
        sb = scatter(g0 + 1, rvb)
        for cp in sa + sb:
            cp.wait()
        return carry

    lax.fori_loop(0, NCH // (2 * FIRE), pair, 0)
    plsc.subcore_barrier()
    pltpu.sync_copy(spacc.at[sl], out_hbm.at[cid].at[sl])


@functools.lru_cache(maxsize=None)
def _sc_kernels():
    mesh = plsc.VectorSubcoreMesh(core_axis_name="c", subcore_axis_name="s")
    params = pltpu.CompilerParams(needs_layout_passes=False,
                                  use_tc_tiling_on_sc=False)
    scalar = pl.kernel(
        _sc_scalar_seg_body,
        mesh=mesh,
        compiler_params=params,
        out_type=jax.ShapeDtypeStruct((NTILES, N), jnp.float32),
        scratch_types=[
            pltpu.VMEM((N,), jnp.float32),
            pltpu.VMEM((EPT,), jnp.int32),
            pltpu.VMEM((EPT,), jnp.int32),
            pltpu.VMEM((N,), jnp.float32),
        ],
    )
    vec = pl.kernel(
        _sc_vec_seg_body,
        mesh=mesh,
        compiler_params=params,
        out_type=jax.ShapeDtypeStruct((2, NP, HIDDEN), jnp.float32),
        scratch_types=[
            pltpu.VMEM((NCH, CH), jnp.int32),
            pltpu.VMEM((NCH, CH), jnp.int32),
            pltpu.VMEM((FIRE * CH, HIDDEN), jnp.float32),
            pltpu.VMEM((FIRE * CH, HIDDEN), jnp.float32),
            pltpu.VMEM_SHARED((NP, HIDDEN), jnp.float32),
            pltpu.SemaphoreType.DMA,
            pltpu.SemaphoreType.DMA,
            pltpu.SemaphoreType.DMA,
        ],
    )
    return scalar, vec


def _topk_mask(score, valid, k, iota_dim):
    imin = jnp.int32(-2147483648)
    b = lax.bitcast_convert_type(score, jnp.int32)
    v = jnp.where(b >= 0, b, imin - b)

    def cnt(pred):
        return jnp.sum(jnp.where(pred & valid, jnp.int32(1), jnp.int32(0)))

    t0 = jnp.where(cnt(v >= 0) >= k, jnp.int32(0), imin)

    def bit_body(i, t):
        tt = t | (jnp.int32(1) << (jnp.int32(30) - i))
        return jnp.where(cnt(v >= tt) >= k, tt, t)

    t = lax.fori_loop(0, 31, bit_body, t0)
    cnt_gt = cnt(v > t)
    need = jnp.int32(k) - cnt_gt
    eq = valid & (v == t)
    idx = lax.broadcasted_iota(jnp.int32, score.shape, iota_dim)

    def ib(i, lohi):
        lo, hi = lohi
        mid = (lo + hi) // 2
        c = jnp.sum(jnp.where(eq & (idx <= mid), jnp.int32(1), jnp.int32(0)))
        ok = c >= need
        return jnp.where(ok, lo, mid + 1), jnp.where(ok, mid, hi)

    lo, _ = lax.fori_loop(0, 14, ib, (jnp.int32(0), jnp.int32(N - 1)))
    return (valid & (v > t)) | (eq & (idx <= lo))


def _eye16():
    r = lax.broadcasted_iota(jnp.int32, (HIDDEN, HIDDEN), 0)
    c = lax.broadcasted_iota(jnp.int32, (HIDDEN, HIDDEN), 1)
    return jnp.where(r == c, 1.0, 0.0).astype(jnp.float32)


def _dg(a, b, ca, cb):
    return lax.dot_general(a, b, (((ca,), (cb,)), ((), ())),
                           preferred_element_type=jnp.float32)


def _to_fmajor(m):
    return _dg(_eye16(), m, 1, 1)


def _to_nmajor(m):
    return _dg(m, _eye16(), 0, 0)


def _tc(f, out_shape, *args):
    return pl.pallas_call(f, out_shape=out_shape)(*args)


def _tc_a(x, W1, degp):
    def f(x_r, w_r, dp_r, hs_r, dinv_r):
        deg = jnp.sum(dp_r[...], axis=0, keepdims=True) + 1.0
        dinv = lax.rsqrt(deg)
        hT = _dg(w_r[...], x_r[...], 0, 1)
        hs_r[...] = _to_nmajor(hT * dinv)
        dinv_r[...] = dinv

    return _tc(f, [jax.ShapeDtypeStruct((N, HIDDEN), jnp.float32),
                   jax.ShapeDtypeStruct((1, N), jnp.float32)],
               x, W1, degp)


def _tc_b(aggp, hs, dinv1, b1, p1r, p1o, p1b):
    def f(ap_r, hs_r, di_r, b1_r, pr_r, po_r, pb_r, h1_r, srel_r, base_r):
        agg = jnp.sum(ap_r[...], axis=0)[:N] + hs_r[...]
        mT = _to_fmajor(agg)
        h1T = jnp.maximum(di_r[...] * mT + b1_r[...], 0.0)
        h1_r[...] = _to_nmajor(h1T)
        srel_r[...] = _dg(pr_r[...], h1T, 0, 0)
        base_r[...] = _dg(po_r[...], h1T, 0, 0) + pb_r[...]

    return _tc(f, [jax.ShapeDtypeStruct((N, HIDDEN), jnp.float32),
                   jax.ShapeDtypeStruct((1, N), jnp.float32),
                   jax.ShapeDtypeStruct((1, N), jnp.float32)],
               aggp, hs, dinv1, b1, p1r, p1o, p1b)


def _tc_c(s1p, base1, h1, W2p):
    def f(sp_r, ba_r, h1_r, w2_r, self_r, z_r):
        score = jnp.sum(sp_r[...], axis=0, keepdims=True) + ba_r[...]
        sel = _topk_mask(score, score > jnp.float32(-jnp.inf), K1, 1)
        m1 = jnp.where(sel, jnp.tanh(score), 0.0)
        xpT = _to_fmajor(h1_r[...]) * m1
        zT = _dg(w2_r[...], xpT, 0, 0)
        self_r[...] = jnp.where(sel, 1.0, 0.0)
        z_r[...] = _to_nmajor(zT)

    return _tc(f, [jax.ShapeDtypeStruct((1, N), jnp.float32),
                   jax.ShapeDtypeStruct((N, HIDDEN), jnp.float32)],
               s1p, base1, h1, W2p)


def _tc_d(d2p, z16):
    def f(dp_r, z_r, zs_r, dinv_r):
        deg2 = jnp.sum(dp_r[...], axis=0, keepdims=True) + 1.0
        dinv2 = lax.rsqrt(deg2)
        zs_r[...] = _to_nmajor(_to_fmajor(z_r[...]) * dinv2)
        dinv_r[...] = dinv2

    return _tc(f, [jax.ShapeDtypeStruct((N, HIDDEN), jnp.float32),
                   jax.ShapeDtypeStruct((1, N), jnp.float32)],
               d2p, z16)


def _tc_e(a2p, zs16, dinv2, b2p, p2r, p2o, p2b, sel1f):
    def f(ap_r, zs_r, di_r, b2_r, pr_r, po_r, pb_r, sf_r, h2_r, srel_r, base_r):
        agg2 = jnp.sum(ap_r[...], axis=0)[:N] + zs_r[...]
        h2T = di_r[...] * _to_fmajor(agg2) + b2_r[...]
        h2_r[...] = h2T
        srel_r[...] = _dg(pr_r[...], h2T, 0, 0) * sf_r[...]
        base_r[...] = _dg(po_r[...], h2T, 0, 0) + pb_r[...]

    return _tc(f, [jax.ShapeDtypeStruct((HIDDEN, N), jnp.float32),
                   jax.ShapeDtypeStruct((1, N), jnp.float32),
                   jax.ShapeDtypeStruct((1, N), jnp.float32)],
               a2p, zs16, dinv2, b2p, p2r, p2o, p2b, sel1f)


def _tc_f(s2p, base2, h2T, sel1f):
    def f(sp_r, ba_r, h2_r, sf_r, out_r):
        score2 = jnp.sum(sp_r[...], axis=0, keepdims=True) + ba_r[...]
        sel2 = _topk_mask(score2, sf_r[...] > 0.5, K2, 1)
        m2 = jnp.where(sel2, jnp.tanh(score2), 0.0)
        gcol = jnp.sum(jnp.where(sel2, h2_r[...] * m2, 0.0), axis=1,
                       keepdims=True)
        grow = _dg(gcol, _eye16(), 0, 0)[:, :NUM_CLASSES] * jnp.float32(1.0 / K2)
        mx = jnp.max(grow, axis=1, keepdims=True)
        sh = grow - mx
        lse = jnp.log(jnp.sum(jnp.exp(sh), axis=1, keepdims=True))
        out_r[...] = sh - lse

    return _tc(f, [jax.ShapeDtypeStruct((1, NUM_CLASSES), jnp.float32)],
               s2p, base2, h2T, sel1f)[0]


def kernel(x, edge_index, batch, W1, b1, W2, b2,
           p1_wrel, p1_wroot, p1_b, p2_wrel, p2_wroot, p2_b):
    row = edge_index[0]
    col = edge_index[1]
    row3 = row.reshape(NTILES, NCH, CH)
    col3 = col.reshape(NTILES, NCH, CH)
    zeros16 = jnp.zeros((NP, HIDDEN), jnp.float32)
    ones_n = jnp.ones((N,), jnp.float32)

    b1r = b1.reshape(HIDDEN, 1)
    W2p = jnp.zeros((HIDDEN, HIDDEN), jnp.float32).at[:, :NUM_CLASSES].set(W2)
    b2p = jnp.zeros((HIDDEN, 1), jnp.float32).at[:NUM_CLASSES, 0].set(b2)
    p2rp = jnp.zeros((HIDDEN, 1), jnp.float32).at[:NUM_CLASSES].set(p2_wrel)
    p2op = jnp.zeros((HIDDEN, 1), jnp.float32).at[:NUM_CLASSES].set(p2_wroot)

    sc_scalar, sc_vec = _sc_kernels()

    degp = sc_scalar(ones_n, row, col)
    hs, dinv1 = _tc_a(x, W1, degp)
    aggp = sc_vec(hs, row3, col3, zeros16)
    h1, s1rel, base1 = _tc_b(aggp, hs, dinv1, b1r, p1_wrel, p1_wroot,
                             p1_b.reshape(1, 1))
    s1p = sc_scalar(s1rel.reshape(N), row, col)
    sel1f, z16 = _tc_c(s1p, base1, h1, W2p)
    d2p = sc_scalar(sel1f.reshape(N), row, col)
    zs16, dinv2 = _tc_d(d2p, z16)
    a2p = sc_vec(zs16, row3, col3, zeros16)
    h2T, s2rel, base2 = _tc_e(a2p, zs16, dinv2, b2p, p2rp, p2op,
                              p2_b.reshape(1, 1), sel1f)
    s2p = sc_scalar(s2rel.reshape(N), row, col)
    return _tc_f(s2p, base2, h2T, sel1f)


# speedup vs baseline: 113.0449x; 1.0438x over previous
import functools

import jax
import jax.numpy as jnp
from jax import lax
from jax.experimental import pallas as pl
from jax.experimental.pallas import tpu as pltpu
from jax.experimental.pallas import tpu_sc as plsc

N = 10000
E = 320000
D_FEAT = 128
HIDDEN = 16
NUM_CLASSES = 10
K1 = 2500
K2 = 625

NP = 10240
SPR = NP // 16
NTILES = 32
EPT = E // NTILES
CH = 125
NCH = EPT // CH
FIRE = 8
GRP = EPT // 16


def _sc_scalar_seg_body(src_hbm, row_hbm, col_hbm, out_hbm, srcv, rbuf, cbuf, acc,
                        sem):
    cid = lax.axis_index("c")
    sid = lax.axis_index("s")
    w = sid * 2 + cid
    base = w * EPT
    cps = [pltpu.async_copy(src_hbm, srcv, sem),
           pltpu.async_copy(row_hbm.at[pl.ds(base, EPT)], rbuf, sem),
           pltpu.async_copy(col_hbm.at[pl.ds(base, EPT)], cbuf, sem)]

    zero = jnp.zeros((16,), jnp.float32)

    def zbody(j5, carry):
        for u in range(5):
            acc[pl.ds((j5 * 5 + u) * 16, 16)] = zero
        return carry

    lax.fori_loop(0, GRP // 5, zbody, 0)
    for cp in cps:
        cp.wait()

    def body(j5, carry):
        for u in range(5):
            j = j5 * 5 + u
            r = rbuf[pl.ds(j * 16, 16)]
            c = cbuf[pl.ds(j * 16, 16)]
            vals = plsc.load_gather(srcv, [r])
            plsc.addupdate_scatter(acc, [c], vals)
        return carry

    lax.fori_loop(0, GRP // 5, body, 0)
    pltpu.sync_copy(acc, out_hbm.at[w])


def _sc_vec_seg_body(rows_hbm, row3_hbm, col3_hbm, zeros_hbm, out_hbm,
                     rbuf, cbuf, rva, rvb, spacc, sga, sgb, ssc):
    cid = lax.axis_index("c")
    sid = lax.axis_index("s")
    w = sid * 2 + cid
    pltpu.sync_copy(row3_hbm.at[w], rbuf)
    pltpu.sync_copy(col3_hbm.at[w], cbuf)
    sl = pl.ds(sid * SPR, SPR)
    pltpu.sync_copy(zeros_hbm.at[sl], spacc.at[sl])
    plsc.subcore_barrier()

    def fire(g, rv, sem):
        return [pltpu.async_copy(rows_hbm.at[rbuf.at[g * FIRE + b]],
                                 rv.at[pl.ds(b * CH, CH)], sem)
                for b in range(FIRE)]

    def scatter(g, rv):
        return [pltpu.async_copy(rv.at[pl.ds(b * CH, CH)],
                                 spacc.at[cbuf.at[g * FIRE + b]], ssc, add=True)
                for b in range(FIRE)]

    def pair(i2, carry):
        g0 = i2 * 2
        ga = fire(g0, rva, sga)
        gb = fire(g0 + 1, rvb, sgb)
        for cp in ga:
            cp.wait()
        sa = scatter(g0, rva)
        for cp in gb:
            cp.wait()
        sb = scatter(g0 + 1, rvb)
        for cp in sa + sb:
            cp.wait()
        return carry

    lax.fori_loop(0, NCH // (2 * FIRE), pair, 0)
    plsc.subcore_barrier()
    pltpu.sync_copy(spacc.at[sl], out_hbm.at[cid].at[sl])


@functools.lru_cache(maxsize=None)
def _sc_kernels():
    mesh = plsc.VectorSubcoreMesh(core_axis_name="c", subcore_axis_name="s")
    params = pltpu.CompilerParams(needs_layout_passes=False,
                                  use_tc_tiling_on_sc=False)
    scalar = pl.kernel(
        _sc_scalar_seg_body,
        mesh=mesh,
        compiler_params=params,
        out_type=jax.ShapeDtypeStruct((NTILES, N), jnp.float32),
        scratch_types=[
            pltpu.VMEM((N,), jnp.float32),
            pltpu.VMEM((EPT,), jnp.int32),
            pltpu.VMEM((EPT,), jnp.int32),
            pltpu.VMEM((N,), jnp.float32),
            pltpu.SemaphoreType.DMA,
        ],
    )
    vec = pl.kernel(
        _sc_vec_seg_body,
        mesh=mesh,
        compiler_params=params,
        out_type=jax.ShapeDtypeStruct((2, NP, HIDDEN), jnp.float32),
        scratch_types=[
            pltpu.VMEM((NCH, CH), jnp.int32),
            pltpu.VMEM((NCH, CH), jnp.int32),
            pltpu.VMEM((FIRE * CH, HIDDEN), jnp.float32),
            pltpu.VMEM((FIRE * CH, HIDDEN), jnp.float32),
            pltpu.VMEM_SHARED((NP, HIDDEN), jnp.float32),
            pltpu.SemaphoreType.DMA,
            pltpu.SemaphoreType.DMA,
            pltpu.SemaphoreType.DMA,
        ],
    )
    return scalar, vec


def _topk_mask(score, valid, k, iota_dim):
    imin = jnp.int32(-2147483648)
    b = lax.bitcast_convert_type(score, jnp.int32)
    v = jnp.where(b >= 0, b, imin - b)

    def cnt(pred):
        return jnp.sum(jnp.where(pred & valid, jnp.int32(1), jnp.int32(0)))

    t0 = jnp.where(cnt(v >= 0) >= k, jnp.int32(0), imin)

    def bit_body(i, t):
        tt = t | (jnp.int32(1) << (jnp.int32(30) - i))
        return jnp.where(cnt(v >= tt) >= k, tt, t)

    t = lax.fori_loop(0, 31, bit_body, t0)
    cnt_gt = cnt(v > t)
    need = jnp.int32(k) - cnt_gt
    eq = valid & (v == t)
    idx = lax.broadcasted_iota(jnp.int32, score.shape, iota_dim)

    def ib(i, lohi):
        lo, hi = lohi
        mid = (lo + hi) // 2
        c = jnp.sum(jnp.where(eq & (idx <= mid), jnp.int32(1), jnp.int32(0)))
        ok = c >= need
        return jnp.where(ok, lo, mid + 1), jnp.where(ok, mid, hi)

    lo, _ = lax.fori_loop(0, 14, ib, (jnp.int32(0), jnp.int32(N - 1)))
    return (valid & (v > t)) | (eq & (idx <= lo))


def _eye16():
    r = lax.broadcasted_iota(jnp.int32, (HIDDEN, HIDDEN), 0)
    c = lax.broadcasted_iota(jnp.int32, (HIDDEN, HIDDEN), 1)
    return jnp.where(r == c, 1.0, 0.0).astype(jnp.float32)


def _dg(a, b, ca, cb):
    return lax.dot_general(a, b, (((ca,), (cb,)), ((), ())),
                           preferred_element_type=jnp.float32)


def _to_fmajor(m):
    return _dg(_eye16(), m, 1, 1)


def _to_nmajor(m):
    return _dg(m, _eye16(), 0, 0)


def _tc(f, out_shape, *args):
    return pl.pallas_call(f, out_shape=out_shape)(*args)


def _tc_a(x, W1, degp):
    def f(x_r, w_r, dp_r, hs_r, dinv_r):
        deg = jnp.sum(dp_r[...], axis=0, keepdims=True) + 1.0
        dinv = lax.rsqrt(deg)
        hT = _dg(w_r[...], x_r[...], 0, 1)
        hs_r[...] = _to_nmajor(hT * dinv)
        dinv_r[...] = dinv

    return _tc(f, [jax.ShapeDtypeStruct((N, HIDDEN), jnp.float32),
                   jax.ShapeDtypeStruct((1, N), jnp.float32)],
               x, W1, degp)


def _tc_b(aggp, hs, dinv1, b1, p1r, p1o, p1b):
    def f(ap_r, hs_r, di_r, b1_r, pr_r, po_r, pb_r, h1_r, srel_r, base_r):
        agg = jnp.sum(ap_r[...], axis=0)[:N] + hs_r[...]
        mT = _to_fmajor(agg)
        h1T = jnp.maximum(di_r[...] * mT + b1_r[...], 0.0)
        h1_r[...] = _to_nmajor(h1T)
        srel_r[...] = _dg(pr_r[...], h1T, 0, 0)
        base_r[...] = _dg(po_r[...], h1T, 0, 0) + pb_r[...]

    return _tc(f, [jax.ShapeDtypeStruct((N, HIDDEN), jnp.float32),
                   jax.ShapeDtypeStruct((1, N), jnp.float32),
                   jax.ShapeDtypeStruct((1, N), jnp.float32)],
               aggp, hs, dinv1, b1, p1r, p1o, p1b)


def _tc_c(s1p, base1, h1, W2p):
    def f(sp_r, ba_r, h1_r, w2_r, self_r, z_r):
        score = jnp.sum(sp_r[...], axis=0, keepdims=True) + ba_r[...]
        sel = _topk_mask(score, score > jnp.float32(-jnp.inf), K1, 1)
        m1 = jnp.where(sel, jnp.tanh(score), 0.0)
        xpT = _to_fmajor(h1_r[...]) * m1
        zT = _dg(w2_r[...], xpT, 0, 0)
        self_r[...] = jnp.where(sel, 1.0, 0.0)
        z_r[...] = _to_nmajor(zT)

    return _tc(f, [jax.ShapeDtypeStruct((1, N), jnp.float32),
                   jax.ShapeDtypeStruct((N, HIDDEN), jnp.float32)],
               s1p, base1, h1, W2p)


def _tc_d(d2p, z16):
    def f(dp_r, z_r, zs_r, dinv_r):
        deg2 = jnp.sum(dp_r[...], axis=0, keepdims=True) + 1.0
        dinv2 = lax.rsqrt(deg2)
        zs_r[...] = _to_nmajor(_to_fmajor(z_r[...]) * dinv2)
        dinv_r[...] = dinv2

    return _tc(f, [jax.ShapeDtypeStruct((N, HIDDEN), jnp.float32),
                   jax.ShapeDtypeStruct((1, N), jnp.float32)],
               d2p, z16)


def _tc_e(a2p, zs16, dinv2, b2p, p2r, p2o, p2b, sel1f):
    def f(ap_r, zs_r, di_r, b2_r, pr_r, po_r, pb_r, sf_r, h2_r, srel_r, base_r):
        agg2 = jnp.sum(ap_r[...], axis=0)[:N] + zs_r[...]
        h2T = di_r[...] * _to_fmajor(agg2) + b2_r[...]
        h2_r[...] = h2T
        srel_r[...] = _dg(pr_r[...], h2T, 0, 0) * sf_r[...]
        base_r[...] = _dg(po_r[...], h2T, 0, 0) + pb_r[...]

    return _tc(f, [jax.ShapeDtypeStruct((HIDDEN, N), jnp.float32),
                   jax.ShapeDtypeStruct((1, N), jnp.float32),
                   jax.ShapeDtypeStruct((1, N), jnp.float32)],
               a2p, zs16, dinv2, b2p, p2r, p2o, p2b, sel1f)


def _tc_f(s2p, base2, h2T, sel1f):
    def f(sp_r, ba_r, h2_r, sf_r, out_r):
        score2 = jnp.sum(sp_r[...], axis=0, keepdims=True) + ba_r[...]
        sel2 = _topk_mask(score2, sf_r[...] > 0.5, K2, 1)
        m2 = jnp.where(sel2, jnp.tanh(score2), 0.0)
        gcol = jnp.sum(jnp.where(sel2, h2_r[...] * m2, 0.0), axis=1,
                       keepdims=True)
        grow = _dg(gcol, _eye16(), 0, 0)[:, :NUM_CLASSES] * jnp.float32(1.0 / K2)
        mx = jnp.max(grow, axis=1, keepdims=True)
        sh = grow - mx
        lse = jnp.log(jnp.sum(jnp.exp(sh), axis=1, keepdims=True))
        out_r[...] = sh - lse

    return _tc(f, [jax.ShapeDtypeStruct((1, NUM_CLASSES), jnp.float32)],
               s2p, base2, h2T, sel1f)[0]


def kernel(x, edge_index, batch, W1, b1, W2, b2,
           p1_wrel, p1_wroot, p1_b, p2_wrel, p2_wroot, p2_b):
    row = edge_index[0]
    col = edge_index[1]
    row3 = row.reshape(NTILES, NCH, CH)
    col3 = col.reshape(NTILES, NCH, CH)
    zeros16 = jnp.zeros((NP, HIDDEN), jnp.float32)
    ones_n = jnp.ones((N,), jnp.float32)

    b1r = b1.reshape(HIDDEN, 1)
    W2p = jnp.zeros((HIDDEN, HIDDEN), jnp.float32).at[:, :NUM_CLASSES].set(W2)
    b2p = jnp.zeros((HIDDEN, 1), jnp.float32).at[:NUM_CLASSES, 0].set(b2)
    p2rp = jnp.zeros((HIDDEN, 1), jnp.float32).at[:NUM_CLASSES].set(p2_wrel)
    p2op = jnp.zeros((HIDDEN, 1), jnp.float32).at[:NUM_CLASSES].set(p2_wroot)

    sc_scalar, sc_vec = _sc_kernels()

    degp = sc_scalar(ones_n, row, col)
    hs, dinv1 = _tc_a(x, W1, degp)
    aggp = sc_vec(hs, row3, col3, zeros16)
    h1, s1rel, base1 = _tc_b(aggp, hs, dinv1, b1r, p1_wrel, p1_wroot,
                             p1_b.reshape(1, 1))
    s1p = sc_scalar(s1rel.reshape(N), row, col)
    sel1f, z16 = _tc_c(s1p, base1, h1, W2p)
    d2p = sc_scalar(sel1f.reshape(N), row, col)
    zs16, dinv2 = _tc_d(d2p, z16)
    a2p = sc_vec(zs16, row3, col3, zeros16)
    h2T, s2rel, base2 = _tc_e(a2p, zs16, dinv2, b2p, p2rp, p2op,
                              p2_b.reshape(1, 1), sel1f)
    s2p = sc_scalar(s2rel.reshape(N), row, col)
    return _tc_f(s2p, base2, h2T, sel1f)


# speedup vs baseline: 114.4385x; 1.0123x over previous
import functools

import jax
import jax.numpy as jnp
from jax import lax
from jax.experimental import pallas as pl
from jax.experimental.pallas import tpu as pltpu
from jax.experimental.pallas import tpu_sc as plsc

N = 10000
E = 320000
D_FEAT = 128
HIDDEN = 16
NUM_CLASSES = 10
K1 = 2500
K2 = 625

NP = 10240
SPR = NP // 16
NTILES = 32
EPT = E // NTILES
CH = 125
NCH = EPT // CH
FIRE = 8
GRP = EPT // 16


def _sc_scalar_seg_body(src_hbm, row_hbm, col_hbm, out_hbm, srcv, rbuf, cbuf, acc,
                        sem):
    cid = lax.axis_index("c")
    sid = lax.axis_index("s")
    w = sid * 2 + cid
    base = w * EPT
    cps = [pltpu.async_copy(src_hbm, srcv, sem),
           pltpu.async_copy(row_hbm.at[pl.ds(base, EPT)], rbuf, sem),
           pltpu.async_copy(col_hbm.at[pl.ds(base, EPT)], cbuf, sem)]

    zero = jnp.zeros((16,), jnp.float32)

    def zbody(j5, carry):
        for u in range(5):
            acc[pl.ds((j5 * 5 + u) * 16, 16)] = zero
        return carry

    lax.fori_loop(0, GRP // 5, zbody, 0)
    for cp in cps:
        cp.wait()

    def body(j5, carry):
        for u in range(5):
            j = j5 * 5 + u
            r = rbuf[pl.ds(j * 16, 16)]
            c = cbuf[pl.ds(j * 16, 16)]
            vals = plsc.load_gather(srcv, [r])
            plsc.addupdate_scatter(acc, [c], vals)
        return carry

    lax.fori_loop(0, GRP // 5, body, 0)
    pltpu.sync_copy(acc, out_hbm.at[w])


def _sc_vec_seg_body(rows_hbm, row3_hbm, col3_hbm, zeros_hbm, out_hbm,
                     rbuf, cbuf, rva, rvb, spacc, sga, sgb, ssc):
    cid = lax.axis_index("c")
    sid = lax.axis_index("s")
    w = sid * 2 + cid
    sl = pl.ds(sid * SPR, SPR)
    cps = [pltpu.async_copy(row3_hbm.at[w], rbuf, sga),
           pltpu.async_copy(col3_hbm.at[w], cbuf, sgb),
           pltpu.async_copy(zeros_hbm.at[sl], spacc.at[sl], ssc)]
    for cp in cps:
        cp.wait()
    plsc.subcore_barrier()

    def fire(g, rv, sem):
        return [pltpu.async_copy(rows_hbm.at[rbuf.at[g * FIRE + b]],
                                 rv.at[pl.ds(b * CH, CH)], sem)
                for b in range(FIRE)]

    def scatter(g, rv):
        return [pltpu.async_copy(rv.at[pl.ds(b * CH, CH)],
                                 spacc.at[cbuf.at[g * FIRE + b]], ssc, add=True)
                for b in range(FIRE)]

    def pair(i2, carry):
        g0 = i2 * 2
        ga = fire(g0, rva, sga)
        gb = fire(g0 + 1, rvb, sgb)
        for cp in ga:
            cp.wait()
        sa = scatter(g0, rva)
        for cp in gb:
            cp.wait()
        sb = scatter(g0 + 1, rvb)
        for cp in sa + sb:
            cp.wait()
        return carry

    lax.fori_loop(0, NCH // (2 * FIRE), pair, 0)
    plsc.subcore_barrier()
    pltpu.sync_copy(spacc.at[sl], out_hbm.at[cid].at[sl])


@functools.lru_cache(maxsize=None)
def _sc_kernels():
    mesh = plsc.VectorSubcoreMesh(core_axis_name="c", subcore_axis_name="s")
    params = pltpu.CompilerParams(needs_layout_passes=False,
                                  use_tc_tiling_on_sc=False)
    scalar = pl.kernel(
        _sc_scalar_seg_body,
        mesh=mesh,
        compiler_params=params,
        out_type=jax.ShapeDtypeStruct((NTILES, N), jnp.float32),
        scratch_types=[
            pltpu.VMEM((N,), jnp.float32),
            pltpu.VMEM((EPT,), jnp.int32),
            pltpu.VMEM((EPT,), jnp.int32),
            pltpu.VMEM((N,), jnp.float32),
            pltpu.SemaphoreType.DMA,
        ],
    )
    vec = pl.kernel(
        _sc_vec_seg_body,
        mesh=mesh,
        compiler_params=params,
        out_type=jax.ShapeDtypeStruct((2, NP, HIDDEN), jnp.float32),
        scratch_types=[
            pltpu.VMEM((NCH, CH), jnp.int32),
            pltpu.VMEM((NCH, CH), jnp.int32),
            pltpu.VMEM((FIRE * CH, HIDDEN), jnp.float32),
            pltpu.VMEM((FIRE * CH, HIDDEN), jnp.float32),
            pltpu.VMEM_SHARED((NP, HIDDEN), jnp.float32),
            pltpu.SemaphoreType.DMA,
            pltpu.SemaphoreType.DMA,
            pltpu.SemaphoreType.DMA,
        ],
    )
    return scalar, vec


def _topk_mask(score, valid, k, iota_dim):
    imin = jnp.int32(-2147483648)
    b = lax.bitcast_convert_type(score, jnp.int32)
    v = jnp.where(b >= 0, b, imin - b)

    def cnt(pred):
        return jnp.sum(jnp.where(pred & valid, jnp.int32(1), jnp.int32(0)))

    t0 = jnp.where(cnt(v >= 0) >= k, jnp.int32(0), imin)

    def bit_body(i, t):
        tt = t | (jnp.int32(1) << (jnp.int32(30) - i))
        return jnp.where(cnt(v >= tt) >= k, tt, t)

    t = lax.fori_loop(0, 31, bit_body, t0)
    cnt_gt = cnt(v > t)
    need = jnp.int32(k) - cnt_gt
    eq = valid & (v == t)
    idx = lax.broadcasted_iota(jnp.int32, score.shape, iota_dim)

    def ib(i, lohi):
        lo, hi = lohi
        mid = (lo + hi) // 2
        c = jnp.sum(jnp.where(eq & (idx <= mid), jnp.int32(1), jnp.int32(0)))
        ok = c >= need
        return jnp.where(ok, lo, mid + 1), jnp.where(ok, mid, hi)

    lo, _ = lax.fori_loop(0, 14, ib, (jnp.int32(0), jnp.int32(N - 1)))
    return (valid & (v > t)) | (eq & (idx <= lo))


def _eye16():
    r = lax.broadcasted_iota(jnp.int32, (HIDDEN, HIDDEN), 0)
    c = lax.broadcasted_iota(jnp.int32, (HIDDEN, HIDDEN), 1)
    return jnp.where(r == c, 1.0, 0.0).astype(jnp.float32)


def _dg(a, b, ca, cb):
    return lax.dot_general(a, b, (((ca,), (cb,)), ((), ())),
                           preferred_element_type=jnp.float32)


def _to_fmajor(m):
    return _dg(_eye16(), m, 1, 1)


def _to_nmajor(m):
    return _dg(m, _eye16(), 0, 0)


def _tc(f, out_shape, *args):
    return pl.pallas_call(f, out_shape=out_shape)(*args)


def _tc_a(x, W1, degp):
    def f(x_r, w_r, dp_r, hs_r, dinv_r):
        deg = jnp.sum(dp_r[...], axis=0, keepdims=True) + 1.0
        dinv = lax.rsqrt(deg)
        hT = _dg(w_r[...], x_r[...], 0, 1)
        hs_r[...] = _to_nmajor(hT * dinv)
        dinv_r[...] = dinv

    return _tc(f, [jax.ShapeDtypeStruct((N, HIDDEN), jnp.float32),
                   jax.ShapeDtypeStruct((1, N), jnp.float32)],
               x, W1, degp)


def _tc_b(aggp, hs, dinv1, b1, p1r, p1o, p1b):
    def f(ap_r, hs_r, di_r, b1_r, pr_r, po_r, pb_r, h1_r, srel_r, base_r):
        agg = jnp.sum(ap_r[...], axis=0)[:N] + hs_r[...]
        mT = _to_fmajor(agg)
        h1T = jnp.maximum(di_r[...] * mT + b1_r[...], 0.0)
        h1_r[...] = _to_nmajor(h1T)
        srel_r[...] = _dg(pr_r[...], h1T, 0, 0)
        base_r[...] = _dg(po_r[...], h1T, 0, 0) + pb_r[...]

    return _tc(f, [jax.ShapeDtypeStruct((N, HIDDEN), jnp.float32),
                   jax.ShapeDtypeStruct((1, N), jnp.float32),
                   jax.ShapeDtypeStruct((1, N), jnp.float32)],
               aggp, hs, dinv1, b1, p1r, p1o, p1b)


def _tc_c(s1p, base1, h1, W2p):
    def f(sp_r, ba_r, h1_r, w2_r, self_r, z_r):
        score = jnp.sum(sp_r[...], axis=0, keepdims=True) + ba_r[...]
        sel = _topk_mask(score, score > jnp.float32(-jnp.inf), K1, 1)
        m1 = jnp.where(sel, jnp.tanh(score), 0.0)
        xpT = _to_fmajor(h1_r[...]) * m1
        zT = _dg(w2_r[...], xpT, 0, 0)
        self_r[...] = jnp.where(sel, 1.0, 0.0)
        z_r[...] = _to_nmajor(zT)

    return _tc(f, [jax.ShapeDtypeStruct((1, N), jnp.float32),
                   jax.ShapeDtypeStruct((N, HIDDEN), jnp.float32)],
               s1p, base1, h1, W2p)


def _tc_d(d2p, z16):
    def f(dp_r, z_r, zs_r, dinv_r):
        deg2 = jnp.sum(dp_r[...], axis=0, keepdims=True) + 1.0
        dinv2 = lax.rsqrt(deg2)
        zs_r[...] = _to_nmajor(_to_fmajor(z_r[...]) * dinv2)
        dinv_r[...] = dinv2

    return _tc(f, [jax.ShapeDtypeStruct((N, HIDDEN), jnp.float32),
                   jax.ShapeDtypeStruct((1, N), jnp.float32)],
               d2p, z16)


def _tc_e(a2p, zs16, dinv2, b2p, p2r, p2o, p2b, sel1f):
    def f(ap_r, zs_r, di_r, b2_r, pr_r, po_r, pb_r, sf_r, h2_r, srel_r, base_r):
        agg2 = jnp.sum(ap_r[...], axis=0)[:N] + zs_r[...]
        h2T = di_r[...] * _to_fmajor(agg2) + b2_r[...]
        h2_r[...] = h2T
        srel_r[...] = _dg(pr_r[...], h2T, 0, 0) * sf_r[...]
        base_r[...] = _dg(po_r[...], h2T, 0, 0) + pb_r[...]

    return _tc(f, [jax.ShapeDtypeStruct((HIDDEN, N), jnp.float32),
                   jax.ShapeDtypeStruct((1, N), jnp.float32),
                   jax.ShapeDtypeStruct((1, N), jnp.float32)],
               a2p, zs16, dinv2, b2p, p2r, p2o, p2b, sel1f)


def _tc_f(s2p, base2, h2T, sel1f):
    def f(sp_r, ba_r, h2_r, sf_r, out_r):
        score2 = jnp.sum(sp_r[...], axis=0, keepdims=True) + ba_r[...]
        sel2 = _topk_mask(score2, sf_r[...] > 0.5, K2, 1)
        m2 = jnp.where(sel2, jnp.tanh(score2), 0.0)
        gcol = jnp.sum(jnp.where(sel2, h2_r[...] * m2, 0.0), axis=1,
                       keepdims=True)
        grow = _dg(gcol, _eye16(), 0, 0)[:, :NUM_CLASSES] * jnp.float32(1.0 / K2)
        mx = jnp.max(grow, axis=1, keepdims=True)
        sh = grow - mx
        lse = jnp.log(jnp.sum(jnp.exp(sh), axis=1, keepdims=True))
        out_r[...] = sh - lse

    return _tc(f, [jax.ShapeDtypeStruct((1, NUM_CLASSES), jnp.float32)],
               s2p, base2, h2T, sel1f)[0]


def kernel(x, edge_index, batch, W1, b1, W2, b2,
           p1_wrel, p1_wroot, p1_b, p2_wrel, p2_wroot, p2_b):
    row = edge_index[0]
    col = edge_index[1]
    row3 = row.reshape(NTILES, NCH, CH)
    col3 = col.reshape(NTILES, NCH, CH)
    zeros16 = jnp.zeros((NP, HIDDEN), jnp.float32)
    ones_n = jnp.ones((N,), jnp.float32)

    b1r = b1.reshape(HIDDEN, 1)
    W2p = jnp.zeros((HIDDEN, HIDDEN), jnp.float32).at[:, :NUM_CLASSES].set(W2)
    b2p = jnp.zeros((HIDDEN, 1), jnp.float32).at[:NUM_CLASSES, 0].set(b2)
    p2rp = jnp.zeros((HIDDEN, 1), jnp.float32).at[:NUM_CLASSES].set(p2_wrel)
    p2op = jnp.zeros((HIDDEN, 1), jnp.float32).at[:NUM_CLASSES].set(p2_wroot)

    sc_scalar, sc_vec = _sc_kernels()

    degp = sc_scalar(ones_n, row, col)
    hs, dinv1 = _tc_a(x, W1, degp)
    aggp = sc_vec(hs, row3, col3, zeros16)
    h1, s1rel, base1 = _tc_b(aggp, hs, dinv1, b1r, p1_wrel, p1_wroot,
                             p1_b.reshape(1, 1))
    s1p = sc_scalar(s1rel.reshape(N), row, col)
    sel1f, z16 = _tc_c(s1p, base1, h1, W2p)
    d2p = sc_scalar(sel1f.reshape(N), row, col)
    zs16, dinv2 = _tc_d(d2p, z16)
    a2p = sc_vec(zs16, row3, col3, zeros16)
    h2T, s2rel, base2 = _tc_e(a2p, zs16, dinv2, b2p, p2rp, p2op,
                              p2_b.reshape(1, 1), sel1f)
    s2p = sc_scalar(s2rel.reshape(N), row, col)
    return _tc_f(s2p, base2, h2T, sel1f)
